# baseline (speedup 1.0000x reference)
import functools
import math

import jax
import jax.numpy as jnp
from jax import lax
from jax.experimental import pallas as pl
from jax.experimental.pallas import tpu as pltpu
from jax.experimental.pallas import tpu_sc as plsc

_DM = 64
_SCALE = math.sqrt(_DM)
_CHUNK = 512
_GSUB = 128
_LANES = 16


@functools.cache
def _make_lookup(n_idx: int, vocab: int):
    info = plsc.get_sparse_core_info()
    nc, ns = info.num_cores, info.num_subcores
    nw = nc * ns
    per_w = n_idx // nw
    assert per_w * nw == n_idx and per_w % _CHUNK == 0
    nchunk = per_w // _CHUNK
    kg = _CHUNK // _GSUB
    rows_per_w_128 = per_w // _GSUB

    mesh = plsc.VectorSubcoreMesh(core_axis_name="c", subcore_axis_name="s")

    @functools.partial(
        pl.kernel,
        mesh=mesh,
        out_type=jax.ShapeDtypeStruct((n_idx, _DM), jnp.float32),
        scratch_types=[
            pltpu.VMEM((kg, _GSUB), jnp.int32),
            pltpu.VMEM((_CHUNK, _DM), jnp.float32),
            pltpu.SemaphoreType.DMA,
        ],
        compiler_params=pltpu.CompilerParams(use_tc_tiling_on_sc=False),
    )
    def lookup(idx_hbm, table_hbm, out_hbm, idx_v, rows_v, gsem):
        wid = lax.axis_index("s") * nc + lax.axis_index("c")

        def chunk_body(g, _):
            row_base = wid * per_w + g * _CHUNK
            idx_row = wid * rows_per_w_128 + g * kg
            pltpu.sync_copy(idx_hbm.at[pl.ds(idx_row, kg)], idx_v)
            cps = [
                pltpu.make_async_copy(
                    table_hbm.at[idx_v.at[j]],
                    rows_v.at[pl.ds(j * _GSUB, _GSUB)],
                    gsem,
                )
                for j in range(kg)
            ]
            for cp in cps:
                cp.start()
            for cp in cps:
                cp.wait()

            unroll = 8

            def scale_body(r0, _):
                r = r0 * unroll
                for u in range(unroll):
                    for v in range(_DM // _LANES):
                        sl = pl.ds(v * _LANES, _LANES)
                        rows_v[r + u, sl] = rows_v[r + u, sl] * _SCALE
                return 0

            lax.fori_loop(0, _CHUNK // unroll, scale_body, 0)
            pltpu.sync_copy(rows_v, out_hbm.at[pl.ds(row_base, _CHUNK)])
            return 0

        lax.fori_loop(0, nchunk, chunk_body, 0)

    return lookup


def kernel(x, table):
    n = x.size
    idx = x.reshape(n // _GSUB, _GSUB)
    out = _make_lookup(n, table.shape[0])(idx, table)
    return out.reshape(x.shape + (table.shape[1],))


# speedup vs baseline: 1.0931x; 1.0931x over previous
import functools
import math

import jax
import jax.numpy as jnp
from jax import lax
from jax.experimental import pallas as pl
from jax.experimental.pallas import tpu as pltpu
from jax.experimental.pallas import tpu_sc as plsc

_DM = 64
_SCALE = math.sqrt(_DM)
_CHUNK = 512
_GSUB = 128
_LANES = 16
_UNROLL = 8


@functools.cache
def _make_lookup(n_idx: int, vocab: int):
    info = plsc.get_sparse_core_info()
    nc, ns = info.num_cores, info.num_subcores
    nw = nc * ns
    per_w = n_idx // nw
    assert per_w * nw == n_idx and per_w % _CHUNK == 0
    nchunk = per_w // _CHUNK
    assert nchunk >= 4 and nchunk % 2 == 0
    kg = _CHUNK // _GSUB
    rows_per_w_128 = per_w // _GSUB

    mesh = plsc.VectorSubcoreMesh(core_axis_name="c", subcore_axis_name="s")

    @functools.partial(
        pl.kernel,
        mesh=mesh,
        out_type=jax.ShapeDtypeStruct((n_idx, _DM), jnp.float32),
        scratch_types=[
            pltpu.VMEM((2, kg, _GSUB), jnp.int32),
            pltpu.VMEM((2, _CHUNK, _DM), jnp.float32),
            pltpu.SemaphoreType.DMA,
            pltpu.SemaphoreType.DMA,
            pltpu.SemaphoreType.DMA,
            pltpu.SemaphoreType.DMA,
            pltpu.SemaphoreType.DMA,
            pltpu.SemaphoreType.DMA,
        ],
        compiler_params=pltpu.CompilerParams(use_tc_tiling_on_sc=False),
    )
    def lookup(idx_hbm, table_hbm, out_hbm, idx_v, rows_v, isem0, isem1,
               gsem0, gsem1, osem0, osem1):
        wid = lax.axis_index("s") * nc + lax.axis_index("c")
        isem = (isem0, isem1)
        gsem = (gsem0, gsem1)
        osem = (osem0, osem1)

        def idx_cp(c, s):
            return pltpu.make_async_copy(
                idx_hbm.at[pl.ds(wid * rows_per_w_128 + c * kg, kg)],
                idx_v.at[s], isem[s])

        def fire_gathers(s):
            cps = [
                pltpu.make_async_copy(
                    table_hbm.at[idx_v.at[s, j]],
                    rows_v.at[s, pl.ds(j * _GSUB, _GSUB)],
                    gsem[s])
                for j in range(kg)
            ]
            for cp in cps:
                cp.start()
            return cps

        def wait_gathers(s):
            for j in range(kg):
                pltpu.make_async_copy(
                    table_hbm.at[idx_v.at[s, j]],
                    rows_v.at[s, pl.ds(j * _GSUB, _GSUB)],
                    gsem[s]).wait()

        def scale(s):
            def body(r0, _):
                r = r0 * _UNROLL
                for u in range(_UNROLL):
                    for v in range(_DM // _LANES):
                        sl = pl.ds(v * _LANES, _LANES)
                        rows_v[s, r + u, sl] = rows_v[s, r + u, sl] * _SCALE
                return 0
            lax.fori_loop(0, _CHUNK // _UNROLL, body, 0)

        def out_cp(c, s):
            return pltpu.make_async_copy(
                rows_v.at[s],
                out_hbm.at[pl.ds(wid * per_w + c * _CHUNK, _CHUNK)],
                osem[s])

        pltpu.sync_copy(idx_hbm.at[pl.ds(wid * rows_per_w_128, kg)],
                        idx_v.at[0])
        fire_gathers(0)
        idx_cp(1, 1).start()
        idx_cp(1, 1).wait()
        fire_gathers(1)
        wait_gathers(0)
        idx_cp(2, 0).start()
        scale(0)
        out_cp(0, 0).start()

        def pair_body(i, _):
            for s, off in ((0, 2), (1, 3)):
                c = 2 * i + off
                idx_cp(c, s).wait()
                out_cp(c, s).wait()
                fire_gathers(s)
                wait_gathers(1 - s)
                nxt = c + 1 if s == 0 else jnp.minimum(c + 1, nchunk - 1)
                idx_cp(nxt, 1 - s).start()
                scale(1 - s)
                out_cp(c - 1, 1 - s).start()
            return 0

        lax.fori_loop(0, (nchunk - 2) // 2, pair_body, 0)

        wait_gathers(1)
        scale(1)
        out_cp(nchunk - 1, 1).start()
        out_cp(nchunk - 2, 0).wait()
        out_cp(nchunk - 1, 1).wait()
        idx_cp(nchunk - 1, 0).wait()

    return lookup


def kernel(x, table):
    n = x.size
    idx = x.reshape(n // _GSUB, _GSUB)
    out = _make_lookup(n, table.shape[0])(idx, table)
    return out.reshape(x.shape + (table.shape[1],))
